# baseline (speedup 1.0000x reference)
import functools

import jax
import jax.numpy as jnp
from jax import lax
from jax.experimental import pallas as pl
from jax.experimental.pallas import tpu as pltpu
from jax.experimental.pallas import tpu_sc as plsc

N_POI = 10000
HID = 256
N_EDGES = 160000
NTILE = 16
NW = 2 * NTILE
APAD = 10240
ROWS = APAD // NW
ECHK = 2048
NCHK = 79
EPAD = ECHK * NCHK
CAP = 8192
G2 = 32
NEG_SLOPE = 0.01
PAD_DST = 1 << 30

_SC_PARAMS = pltpu.CompilerParams(
    use_tc_tiling_on_sc=False, needs_layout_passes=False)


def _make_bucket():
    mesh = plsc.VectorSubcoreMesh(core_axis_name="c", subcore_axis_name="s")

    @functools.partial(
        pl.kernel,
        mesh=mesh,
        compiler_params=_SC_PARAMS,
        out_type=[
            jax.ShapeDtypeStruct((NW, CAP), jnp.int32),
            jax.ShapeDtypeStruct((NW, CAP), jnp.int32),
            jax.ShapeDtypeStruct((NW, CAP), jnp.float32),
            jax.ShapeDtypeStruct((NW, 16), jnp.int32),
        ],
        scratch_types=[
            pltpu.VMEM((ECHK,), jnp.int32),
            pltpu.VMEM((ECHK,), jnp.int32),
            pltpu.VMEM((ECHK,), jnp.float32),
            pltpu.VMEM((CAP,), jnp.int32),
            pltpu.VMEM((CAP,), jnp.int32),
            pltpu.VMEM((CAP,), jnp.float32),
            pltpu.VMEM((16,), jnp.int32),
        ],
    )
    def bucket(src_hbm, dst_hbm, dist_hbm,
               csrc_hbm, cdst_hbm, cw_hbm, ccnt_hbm,
               sb, db, fb, csrcb, cdstb, cwb, cntb):
        c = lax.axis_index("c")
        s = lax.axis_index("s")
        wid = s * 2 + c
        lo = wid * ROWS

        def chunk_body(ch, cnt):
            base = ch * ECHK
            pltpu.sync_copy(src_hbm.at[pl.ds(base, ECHK)], sb)
            pltpu.sync_copy(dst_hbm.at[pl.ds(base, ECHK)], db)
            pltpu.sync_copy(dist_hbm.at[pl.ds(base, ECHK)], fb)

            def grp(i, cnt):
                sl = pl.ds(i * 16, 16)
                d = db[sl]
                m = (d >= lo) & (d < lo + ROWS)
                pos = cnt + plsc.cumsum(jnp.where(m, 1, 0)) - 1
                dist16 = fb[sl]
                plsc.store_scatter(csrcb, [pos], sb[sl], mask=m)
                plsc.store_scatter(cdstb, [pos], d - lo, mask=m)
                plsc.store_scatter(cwb, [pos], jnp.exp(-(dist16 * dist16)),
                                   mask=m)
                npos = plsc.all_reduce_population_count(m)[0]
                return jnp.minimum(cnt + npos, CAP - 64)
            return lax.fori_loop(0, ECHK // 16, grp, cnt)

        cnt = lax.fori_loop(0, NCHK, chunk_body, jnp.int32(0))

        iot = lax.iota(jnp.int32, 16)
        for t in range(4):
            idx = cnt + t * 16 + iot
            plsc.store_scatter(csrcb, [idx], jnp.zeros((16,), jnp.int32))
            plsc.store_scatter(cdstb, [idx], jnp.zeros((16,), jnp.int32))
            plsc.store_scatter(cwb, [idx], jnp.zeros((16,), jnp.float32))

        cntb[:] = jnp.full((16,), 0, jnp.int32) + cnt
        pltpu.sync_copy(csrcb, csrc_hbm.at[wid])
        pltpu.sync_copy(cdstb, cdst_hbm.at[wid])
        pltpu.sync_copy(cwb, cw_hbm.at[wid])
        pltpu.sync_copy(cntb, ccnt_hbm.at[wid])

    return bucket


def _make_propagate():
    mesh = plsc.VectorSubcoreMesh(core_axis_name="c", subcore_axis_name="s")

    @functools.partial(
        pl.kernel,
        mesh=mesh,
        compiler_params=_SC_PARAMS,
        out_type=jax.ShapeDtypeStruct((APAD, HID), jnp.float32),
        scratch_types=[
            pltpu.VMEM((ROWS, HID), jnp.float32),
            pltpu.VMEM((CAP,), jnp.int32),
            pltpu.VMEM((CAP,), jnp.int32),
            pltpu.VMEM((CAP,), jnp.float32),
            pltpu.VMEM((16,), jnp.int32),
            [pltpu.VMEM((G2, HID), jnp.float32) for _ in range(2)],
            [pltpu.SemaphoreType.DMA for _ in range(2)],
        ],
    )
    def propagate(x_hbm, csrc_hbm, cdst_hbm, cw_hbm, ccnt_hbm, out_hbm,
                  accb, csrcb, cdstb, cwb, cntb, gbufs, gsems):
        c = lax.axis_index("c")
        s = lax.axis_index("s")
        wid = s * 2 + c

        pltpu.sync_copy(csrc_hbm.at[wid], csrcb)
        pltpu.sync_copy(cdst_hbm.at[wid], cdstb)
        pltpu.sync_copy(cw_hbm.at[wid], cwb)
        pltpu.sync_copy(ccnt_hbm.at[wid], cntb)
        n = cntb[:][0]
        ngrp = (n + (G2 - 1)) // G2

        zero16 = jnp.zeros((16,), jnp.float32)

        def zrow(r, carry):
            for j in range(HID // 16):
                accb[r, pl.ds(j * 16, 16)] = zero16
            return carry
        lax.fori_loop(0, ROWS, zrow, 0)

        def start_group(g, b):
            pltpu.async_copy(
                x_hbm.at[csrcb.at[pl.ds(g * G2, G2)]], gbufs[b], gsems[b])

        def process_group(g, b):
            pltpu.make_async_copy(
                x_hbm.at[csrcb.at[pl.ds(g * G2, G2)]], gbufs[b],
                gsems[b]).wait()
            gb = gbufs[b]
            for k in range(G2 // 16):
                sl16 = pl.ds(g * G2 + k * 16, 16)
                d16 = cdstb[sl16]
                w16 = cwb[sl16]
                for e in range(16):
                    dl = d16[e]
                    w = w16[e]
                    row = gb.at[k * 16 + e]
                    arow = accb.at[dl]
                    for j in range(HID // 16):
                        sl = pl.ds(j * 16, 16)
                        plsc.addupdate(arow.at[sl], row[sl] * w)

        @pl.when(ngrp > 0)
        def _():
            start_group(0, 0)

        @pl.when(ngrp > 1)
        def _():
            start_group(1, 1)

        def pair_body(i, carry):
            for k in range(2):
                g = i * 2 + k

                @pl.when(g < ngrp)
                def _():
                    process_group(g, k)

                    @pl.when(g + 2 < ngrp)
                    def _():
                        start_group(g + 2, k)
            return carry
        lax.fori_loop(0, (ngrp + 1) // 2, pair_body, 0)

        pltpu.sync_copy(accb, out_hbm.at[pl.ds(wid * ROWS, ROWS)])

    return propagate


_bucket = _make_bucket()
_propagate = _make_propagate()

_TC_ROWS = 1000


def _tc_layer(S, X, W, b, acc, *, scale):
    def body(s_ref, x_ref, w_ref, b_ref, a_ref, y_ref, aout_ref):
        h = s_ref[...] + x_ref[...]
        z = jnp.dot(h, w_ref[...], preferred_element_type=jnp.float32) + b_ref[...]
        y = jnp.where(z >= 0, z, NEG_SLOPE * z)
        y_ref[...] = y
        aout_ref[...] = (a_ref[...] + y) * scale

    return pl.pallas_call(
        body,
        grid=(N_POI // _TC_ROWS,),
        in_specs=[
            pl.BlockSpec((_TC_ROWS, HID), lambda i: (i, 0)),
            pl.BlockSpec((_TC_ROWS, HID), lambda i: (i, 0)),
            pl.BlockSpec((HID, HID), lambda i: (0, 0)),
            pl.BlockSpec((1, HID), lambda i: (0, 0)),
            pl.BlockSpec((_TC_ROWS, HID), lambda i: (i, 0)),
        ],
        out_specs=[
            pl.BlockSpec((_TC_ROWS, HID), lambda i: (i, 0)),
            pl.BlockSpec((_TC_ROWS, HID), lambda i: (i, 0)),
        ],
        out_shape=[
            jax.ShapeDtypeStruct((N_POI, HID), jnp.float32),
            jax.ShapeDtypeStruct((N_POI, HID), jnp.float32),
        ],
    )(S, X, W, b, acc)


def kernel(poi_embs, edge_index, dist, W0, b0, W1, b1, W2, b2):
    src = edge_index[0].astype(jnp.int32)
    dst = edge_index[1].astype(jnp.int32)
    pad = EPAD - N_EDGES
    src_p = jnp.concatenate([src, jnp.zeros((pad,), jnp.int32)])
    dst_p = jnp.concatenate([dst, jnp.full((pad,), PAD_DST, jnp.int32)])
    dist_p = jnp.concatenate(
        [dist.astype(jnp.float32), jnp.zeros((pad,), jnp.float32)])

    csrc, cdst, cw, ccnt = _bucket(src_p, dst_p, dist_p)

    X = poi_embs
    acc = X
    for l, (W, b) in enumerate(((W0, b0), (W1, b1), (W2, b2))):
        S = _propagate(X, csrc, cdst, cw, ccnt)
        X, acc = _tc_layer(S, X, W, b.reshape(1, HID), acc,
                           scale=(0.25 if l == 2 else 1.0))
    return acc
